# baseline (speedup 1.0000x reference)
import functools

import jax
import jax.numpy as jnp
import numpy as np
from jax import lax
from jax.experimental import pallas as pl
from jax.experimental.pallas import tpu as pltpu
from jax.experimental.pallas import tpu_sc as plsc

_DIM = 64
_SCALE = np.float32(8.0)
_LANES = 16


@functools.lru_cache(maxsize=None)
def _build(B, D, NW, C):
    b_per_w = B // NW
    n_chunks = b_per_w // C
    assert b_per_w % C == 0 and B % NW == 0 and C % 8 == 0

    mesh = plsc.VectorSubcoreMesh(core_axis_name="c", subcore_axis_name="s")

    @functools.partial(
        pl.kernel,
        mesh=mesh,
        out_type=jax.ShapeDtypeStruct((B, D), jnp.float32),
        compiler_params=pltpu.CompilerParams(use_tc_tiling_on_sc=False),
        scratch_types=[
            pltpu.VMEM((n_chunks, C), jnp.int32),
            pltpu.VMEM((C, D), jnp.float32),
            pltpu.SemaphoreType.DMA,
        ],
    )
    def gather_scale(idx_hbm, table_hbm, out_hbm, idx_v, buf, gsem):
        wid = lax.axis_index("s") * 2 + lax.axis_index("c")
        base = wid * b_per_w
        pltpu.sync_copy(idx_hbm.at[wid], idx_v)

        def chunk_body(j, carry):
            pltpu.async_copy(table_hbm.at[idx_v.at[j]], buf, gsem).wait()

            def row_body(r, c2):
                for kk in range(D // _LANES):
                    sl = pl.ds(kk * _LANES, _LANES)
                    buf[r, sl] = buf[r, sl] * _SCALE
                return c2

            lax.fori_loop(0, C, row_body, 0, unroll=4)
            pltpu.sync_copy(buf, out_hbm.at[pl.ds(base + j * C, C)])
            return carry

        lax.fori_loop(0, n_chunks, chunk_body, 0)

    return gather_scale


def kernel(input_tensor, table):
    Bt, S = input_tensor.shape
    V, D = table.shape
    B = Bt * S
    NW = 32
    C = 128
    fn = _build(B, D, NW, C)
    idx = input_tensor.reshape(NW, (B // NW) // C, C).astype(jnp.int32)
    out = fn(idx, table)
    return out.reshape(Bt, S, D)
